# baseline (speedup 1.0000x reference)
import jax
import jax.numpy as jnp
from jax import lax
from jax.experimental import pallas as pl
from jax.experimental.pallas import tpu as pltpu
from jax.experimental.pallas import tpu_sc as plsc

N, W, D = 100000, 3, 128
NC, NS, L = 2, 16, 16
NW = NC * NS
C = 32
NBLK = -(-N // C)
BPW, BREM = divmod(NBLK, NW)
NB = 4
ROUNDS = (BPW + 1 + NB - 1) // NB
INV_W = 1.0 / W


def _body(self_hbm, neigh_hbm, out_hbm,
          sbuf0, sbuf1, sbuf2, sbuf3, nbuf0, nbuf1, nbuf2, nbuf3,
          obuf0, obuf1, obuf2, obuf3,
          isem0, isem1, isem2, isem3, osem0, osem1, osem2, osem3):
    sbufs = [sbuf0, sbuf1, sbuf2, sbuf3]
    nbufs = [nbuf0, nbuf1, nbuf2, nbuf3]
    obufs = [obuf0, obuf1, obuf2, obuf3]
    isems = [isem0, isem1, isem2, isem3]
    osems = [osem0, osem1, osem2, osem3]

    wid = lax.axis_index("s") * NC + lax.axis_index("c")
    base_blk = wid * BPW + jnp.minimum(wid, BREM)
    nblk = BPW + jnp.where(wid < BREM, 1, 0)

    def start_in(i, b):
        row0 = jnp.minimum((base_blk + i) * C, N - C)
        for w in range(W):
            pltpu.async_copy(self_hbm.at[w, pl.ds(row0, C)], sbufs[b].at[w],
                             isems[b])
            pltpu.async_copy(neigh_hbm.at[w, pl.ds(row0, C)], nbufs[b].at[w],
                             isems[b])

    def wait_in(b):
        for w in range(W):
            pltpu.make_async_copy(self_hbm.at[w, pl.ds(0, C)], sbufs[b].at[w],
                                  isems[b]).wait()
            pltpu.make_async_copy(neigh_hbm.at[w, pl.ds(0, C)], nbufs[b].at[w],
                                  isems[b]).wait()

    def start_out(i, b):
        row0 = jnp.minimum((base_blk + i) * C, N - C)
        pltpu.async_copy(obufs[b], out_hbm.at[pl.ds(row0, C)], osems[b])

    def wait_out(b):
        pltpu.make_async_copy(obufs[b], out_hbm.at[pl.ds(0, C)], osems[b]).wait()

    def compute(b):
        sbuf, nbuf, obuf = sbufs[b], nbufs[b], obufs[b]

        @plsc.parallel_loop(0, C)
        def rowloop(r):
            vals = []
            for j in range(D // L):
                o = j * L
                vals.append((sbuf[0, r, pl.ds(o, L)], sbuf[1, r, pl.ds(o, L)],
                             sbuf[2, r, pl.ds(o, L)], nbuf[0, r, pl.ds(o, L)],
                             nbuf[1, r, pl.ds(o, L)], nbuf[2, r, pl.ds(o, L)]))
            res = []
            for (s0, s1, s2, n0, n1, n2) in vals:
                acc = ((s0 + s1) + (s2 + n0)) + (n1 + n2)
                res.append(jnp.maximum(acc * jnp.float32(INV_W),
                                       jnp.float32(0.0)))
            for j, v in enumerate(res):
                obuf[r, pl.ds(j * L, L)] = v

    for b in range(NB):
        start_in(b, b)

    def round_(rr, carry):
        for b in range(NB):
            i = rr * NB + b

            @pl.when(i < nblk)
            def _():
                wait_in(b)

                @pl.when(i >= NB)
                def _():
                    wait_out(b)

                compute(b)
                start_out(i, b)

                @pl.when(i + NB < nblk)
                def _():
                    start_in(i + NB, b)

        return carry

    lax.fori_loop(0, ROUNDS, round_, 0)

    for b in range(NB):
        wait_out(b)


@jax.jit
def kernel(self_vecs, neigh_vecs):
    mesh = plsc.VectorSubcoreMesh(core_axis_name="c", subcore_axis_name="s")
    k = pl.kernel(
        _body,
        mesh=mesh,
        compiler_params=pltpu.CompilerParams(use_tc_tiling_on_sc=True),
        out_type=jax.ShapeDtypeStruct((N, D), jnp.float32),
        scratch_types=(
            [pltpu.VMEM((W, C, D), jnp.float32)] * (2 * NB)
            + [pltpu.VMEM((C, D), jnp.float32)] * NB
            + [pltpu.SemaphoreType.DMA] * (2 * NB)
        ),
    )
    return k(jnp.transpose(self_vecs, (1, 0, 2)),
             jnp.transpose(neigh_vecs, (1, 0, 2)))


# speedup vs baseline: 1.0253x; 1.0253x over previous
import jax
import jax.numpy as jnp
from jax import lax
from jax.experimental import pallas as pl
from jax.experimental.pallas import tpu as pltpu
from jax.experimental.pallas import tpu_sc as plsc

N, W, D = 100000, 3, 128
NC, NS, L = 2, 16, 16
NW = NC * NS
C = 40
NBLK = -(-N // C)
BPW, BREM = divmod(NBLK, NW)
NB = 3
ROUNDS = (BPW + 1 + NB - 1) // NB
INV_W = 1.0 / W


def _body(self_hbm, neigh_hbm, out_hbm,
          sbuf0, sbuf1, sbuf2, nbuf0, nbuf1, nbuf2,
          obuf0, obuf1, obuf2,
          isem0, isem1, isem2, osem0, osem1, osem2):
    sbufs = [sbuf0, sbuf1, sbuf2]
    nbufs = [nbuf0, nbuf1, nbuf2]
    obufs = [obuf0, obuf1, obuf2]
    isems = [isem0, isem1, isem2]
    osems = [osem0, osem1, osem2]

    wid = lax.axis_index("s") * NC + lax.axis_index("c")
    base_blk = wid * BPW + jnp.minimum(wid, BREM)
    nblk = BPW + jnp.where(wid < BREM, 1, 0)

    def start_in(i, b):
        row0 = jnp.minimum((base_blk + i) * C, N - C)
        pltpu.async_copy(self_hbm.at[:, pl.ds(row0, C)], sbufs[b], isems[b])
        pltpu.async_copy(neigh_hbm.at[:, pl.ds(row0, C)], nbufs[b], isems[b])

    def wait_in(b):
        pltpu.make_async_copy(self_hbm.at[:, pl.ds(0, C)], sbufs[b],
                              isems[b]).wait()
        pltpu.make_async_copy(neigh_hbm.at[:, pl.ds(0, C)], nbufs[b],
                              isems[b]).wait()

    def start_out(i, b):
        row0 = jnp.minimum((base_blk + i) * C, N - C)
        pltpu.async_copy(obufs[b], out_hbm.at[pl.ds(row0, C)], osems[b])

    def wait_out(b):
        pltpu.make_async_copy(obufs[b], out_hbm.at[pl.ds(0, C)], osems[b]).wait()

    def compute(b):
        sbuf, nbuf, obuf = sbufs[b], nbufs[b], obufs[b]

        @plsc.parallel_loop(0, C)
        def rowloop(r):
            vals = []
            for j in range(D // L):
                o = j * L
                vals.append((sbuf[0, r, pl.ds(o, L)], sbuf[1, r, pl.ds(o, L)],
                             sbuf[2, r, pl.ds(o, L)], nbuf[0, r, pl.ds(o, L)],
                             nbuf[1, r, pl.ds(o, L)], nbuf[2, r, pl.ds(o, L)]))
            res = []
            for (s0, s1, s2, n0, n1, n2) in vals:
                acc = ((s0 + s1) + (s2 + n0)) + (n1 + n2)
                res.append(jnp.maximum(acc * jnp.float32(INV_W),
                                       jnp.float32(0.0)))
            for j, v in enumerate(res):
                obuf[r, pl.ds(j * L, L)] = v

    for b in range(NB):
        start_in(b, b)

    def round_(rr, carry):
        for b in range(NB):
            i = rr * NB + b

            @pl.when(i < nblk)
            def _():
                wait_in(b)

                @pl.when(i >= NB)
                def _():
                    wait_out(b)

                compute(b)
                start_out(i, b)

                @pl.when(i + NB < nblk)
                def _():
                    start_in(i + NB, b)

        return carry

    lax.fori_loop(0, ROUNDS, round_, 0)

    for b in range(NB):
        wait_out(b)


@jax.jit
def kernel(self_vecs, neigh_vecs):
    mesh = plsc.VectorSubcoreMesh(core_axis_name="c", subcore_axis_name="s")
    k = pl.kernel(
        _body,
        mesh=mesh,
        compiler_params=pltpu.CompilerParams(use_tc_tiling_on_sc=True),
        out_type=jax.ShapeDtypeStruct((N, D), jnp.float32),
        scratch_types=(
            [pltpu.VMEM((W, C, D), jnp.float32)] * (2 * NB)
            + [pltpu.VMEM((C, D), jnp.float32)] * NB
            + [pltpu.SemaphoreType.DMA] * (2 * NB)
        ),
    )
    return k(jnp.transpose(self_vecs, (1, 0, 2)),
             jnp.transpose(neigh_vecs, (1, 0, 2)))


# speedup vs baseline: 1.0267x; 1.0014x over previous
import jax
import jax.numpy as jnp
from jax import lax
from jax.experimental import pallas as pl
from jax.experimental.pallas import tpu as pltpu
from jax.experimental.pallas import tpu_sc as plsc

N, W, D = 100000, 3, 128
NC, NS, L = 2, 16, 16
NW = NC * NS
C = 40
NBLK = -(-N // C)
BPW, BREM = divmod(NBLK, NW)
NB = 3
ROUNDS = (BPW + 1 + NB - 1) // NB
INV_W = 1.0 / W


def _body(self_hbm, neigh_hbm, out_hbm,
          sbuf0, sbuf1, sbuf2, nbuf0, nbuf1, nbuf2,
          obuf0, obuf1, obuf2,
          isem0, isem1, isem2, osem0, osem1, osem2):
    sbufs = [sbuf0, sbuf1, sbuf2]
    nbufs = [nbuf0, nbuf1, nbuf2]
    obufs = [obuf0, obuf1, obuf2]
    isems = [isem0, isem1, isem2]
    osems = [osem0, osem1, osem2]

    wid = lax.axis_index("s") * NC + lax.axis_index("c")
    base_blk = wid * BPW + jnp.minimum(wid, BREM)
    nblk = BPW + jnp.where(wid < BREM, 1, 0)

    def start_in(i, b):
        row0 = jnp.minimum((base_blk + i) * C, N - C)
        pltpu.async_copy(self_hbm.at[:, pl.ds(row0, C)], sbufs[b], isems[b])
        pltpu.async_copy(neigh_hbm.at[:, pl.ds(row0, C)], nbufs[b], isems[b])

    def wait_in(b):
        pltpu.make_async_copy(self_hbm.at[:, pl.ds(0, C)], sbufs[b],
                              isems[b]).wait()
        pltpu.make_async_copy(neigh_hbm.at[:, pl.ds(0, C)], nbufs[b],
                              isems[b]).wait()

    def start_out(i, b):
        row0 = jnp.minimum((base_blk + i) * C, N - C)
        pltpu.async_copy(obufs[b], out_hbm.at[pl.ds(row0, C)], osems[b])

    def wait_out(b):
        pltpu.make_async_copy(obufs[b], out_hbm.at[pl.ds(0, C)], osems[b]).wait()

    def compute(b):
        sbuf, nbuf, obuf = sbufs[b], nbufs[b], obufs[b]

        @plsc.parallel_loop(0, C)
        def rowloop(r):
            vals = []
            for j in range(D // L):
                o = j * L
                vals.append((sbuf[0, r, pl.ds(o, L)], sbuf[1, r, pl.ds(o, L)],
                             sbuf[2, r, pl.ds(o, L)], nbuf[0, r, pl.ds(o, L)],
                             nbuf[1, r, pl.ds(o, L)], nbuf[2, r, pl.ds(o, L)]))
            res = []
            for (s0, s1, s2, n0, n1, n2) in vals:
                acc = ((s0 + s1) + (s2 + n0)) + (n1 + n2)
                res.append(jnp.maximum(acc * jnp.float32(INV_W),
                                       jnp.float32(0.0)))
            for j, v in enumerate(res):
                obuf[r, pl.ds(j * L, L)] = v

    for b in range(NB):
        start_in(b, b)

    def round_(rr, carry):
        for b in range(NB):
            i = rr * NB + b

            @pl.when(i < nblk)
            def _():
                wait_in(b)

                @pl.when(i >= NB)
                def _():
                    wait_out(b)

                compute(b)
                start_out(i, b)

                @pl.when(i + NB < nblk)
                def _():
                    start_in(i + NB, b)

        return carry

    lax.fori_loop(0, ROUNDS, round_, 0)

    for b in range(NB):
        wait_out(b)


@jax.jit
def kernel(self_vecs, neigh_vecs):
    mesh = plsc.VectorSubcoreMesh(core_axis_name="c", subcore_axis_name="s")
    k = pl.kernel(
        _body,
        mesh=mesh,
        out_type=jax.ShapeDtypeStruct((N, D), jnp.float32),
        scratch_types=(
            [pltpu.VMEM((W, C, D), jnp.float32)] * (2 * NB)
            + [pltpu.VMEM((C, D), jnp.float32)] * NB
            + [pltpu.SemaphoreType.DMA] * (2 * NB)
        ),
    )
    return k(jnp.transpose(self_vecs, (1, 0, 2)),
             jnp.transpose(neigh_vecs, (1, 0, 2)))
